# baseline (speedup 1.0000x reference)
import functools

import jax
import jax.numpy as jnp
from jax import lax
from jax.experimental import pallas as pl
from jax.experimental.pallas import tpu as pltpu

_A = 16
_B = 3
_BK = 8192


def _body(invt_ref, lT_ref, wT_ref, atoms_ref, o_ref):
    invt = invt_ref[0, 0]
    e = jnp.exp(lT_ref[...] * invt)
    e3 = e.reshape(_B, _A, e.shape[-1])
    s = jnp.sum(e3, axis=1, keepdims=True)
    c = wT_ref[...].reshape(_B, 1, -1) / s
    m = jnp.sum(e3 * c, axis=0)
    a_soft = jnp.tanh(atoms_ref[...] * invt)
    o_ref[...] = lax.dot_general(
        m, a_soft, dimension_numbers=(((0,), (0,)), ((), ())),
        preferred_element_type=jnp.float32)


@functools.partial(jax.jit, static_argnames=("interpret",))
def kernel(atoms, combo_weights, combo_indices_logits, temperature, interpret=False):
    k, b, a = combo_indices_logits.shape
    r = atoms.shape[1]
    invt = (1.0 / jnp.maximum(jnp.asarray(temperature, jnp.float32), 0.1))
    invt = invt.reshape(1, 1)
    lT = combo_indices_logits.transpose(1, 2, 0).reshape(b * a, k)
    wT = combo_weights.T
    grid = (k // _BK,)
    return pl.pallas_call(
        _body,
        grid=grid,
        in_specs=[
            pl.BlockSpec((1, 1), lambda i: (0, 0), memory_space=pltpu.SMEM),
            pl.BlockSpec((b * a, _BK), lambda i: (0, i)),
            pl.BlockSpec((b, _BK), lambda i: (0, i)),
            pl.BlockSpec((a, r), lambda i: (0, 0)),
        ],
        out_specs=pl.BlockSpec((_BK, r), lambda i: (i, 0)),
        out_shape=jax.ShapeDtypeStruct((k, r), jnp.float32),
        interpret=interpret,
    )(invt, lT, wT, atoms)


# speedup vs baseline: 1.1253x; 1.1253x over previous
import functools

import jax
import jax.numpy as jnp
from jax import lax
from jax.experimental import pallas as pl
from jax.experimental.pallas import tpu as pltpu

_A = 16
_B = 3
_BK = 4096


def _body(invt_ref, lT_ref, wT_ref, atoms_ref, o_ref):
    invt = invt_ref[0, 0]
    e = jnp.exp(lT_ref[...] * invt)
    e3 = e.reshape(_B, _A, e.shape[-1])
    s = jnp.sum(e3, axis=1, keepdims=True)
    c = wT_ref[...].reshape(_B, 1, -1) / s
    m = jnp.sum(e3 * c, axis=0)
    a_soft = jnp.tanh(atoms_ref[...] * invt)
    o_ref[...] = lax.dot_general(
        m, a_soft, dimension_numbers=(((0,), (0,)), ((), ())),
        preferred_element_type=jnp.float32)


@functools.partial(jax.jit, static_argnames=("interpret",))
def kernel(atoms, combo_weights, combo_indices_logits, temperature, interpret=False):
    k, b, a = combo_indices_logits.shape
    r = atoms.shape[1]
    invt = (1.0 / jnp.maximum(jnp.asarray(temperature, jnp.float32), 0.1))
    invt = invt.reshape(1, 1)
    lT = combo_indices_logits.transpose(1, 2, 0).reshape(b * a, k)
    wT = combo_weights.T
    grid = (k // _BK,)
    return pl.pallas_call(
        _body,
        grid=grid,
        in_specs=[
            pl.BlockSpec((1, 1), lambda i: (0, 0), memory_space=pltpu.SMEM),
            pl.BlockSpec((b * a, _BK), lambda i: (0, i)),
            pl.BlockSpec((b, _BK), lambda i: (0, i)),
            pl.BlockSpec((a, r), lambda i: (0, 0)),
        ],
        out_specs=pl.BlockSpec((_BK, r), lambda i: (i, 0)),
        out_shape=jax.ShapeDtypeStruct((k, r), jnp.float32),
        interpret=interpret,
    )(invt, lT, wT, atoms)


# speedup vs baseline: 1.1321x; 1.0061x over previous
import jax
import jax.numpy as jnp
from jax import lax
from jax.experimental import pallas as pl
from jax.experimental.pallas import tpu as pltpu

_A = 16
_B = 3
_BK = 4096


def _body(t_ref, lT_ref, wT_ref, atoms_ref, o_ref):
    invt = 1.0 / jnp.maximum(t_ref[0, 0], 0.1)
    e = jnp.exp(lT_ref[...] * invt)
    e3 = e.reshape(_B, _A, e.shape[-1])
    s = jnp.sum(e3, axis=1, keepdims=True)
    c = wT_ref[...].reshape(_B, 1, -1) / s
    m = jnp.sum(e3 * c, axis=0)
    a_soft = jnp.tanh(atoms_ref[...] * invt)
    o_ref[...] = lax.dot_general(
        m, a_soft, dimension_numbers=(((0,), (0,)), ((), ())),
        preferred_element_type=jnp.float32)


@jax.jit
def kernel(atoms, combo_weights, combo_indices_logits, temperature):
    k, b, a = combo_indices_logits.shape
    r = atoms.shape[1]
    t = jnp.asarray(temperature, jnp.float32).reshape(1, 1)
    lT = combo_indices_logits.transpose(1, 2, 0).reshape(b * a, k)
    wT = combo_weights.T
    return pl.pallas_call(
        _body,
        grid=(k // _BK,),
        in_specs=[
            pl.BlockSpec((1, 1), lambda i: (0, 0), memory_space=pltpu.SMEM),
            pl.BlockSpec((b * a, _BK), lambda i: (0, i)),
            pl.BlockSpec((b, _BK), lambda i: (0, i)),
            pl.BlockSpec((a, r), lambda i: (0, 0)),
        ],
        out_specs=pl.BlockSpec((_BK, r), lambda i: (i, 0)),
        out_shape=jax.ShapeDtypeStruct((k, r), jnp.float32),
    )(t, lT, wT, atoms)
